# Initial kernel scaffold; baseline (speedup 1.0000x reference)
import jax
import jax.numpy as jnp
from jax.experimental import pallas as pl


def kernel(x, y, pad_id, embed_x, embed_y):
    raise NotImplementedError("write your pallas kernel here")


# speedup vs baseline: 1.5486x; 1.5486x over previous
import functools

import jax
import jax.numpy as jnp
from jax import lax
from jax.experimental import pallas as pl
from jax.experimental.pallas import tpu as pltpu
from jax.experimental.pallas import tpu_sc as plsc

B = 4096
L = 200
LP = L + 1
D = 32
NC = 2
NS = 16
NW = NC * NS
BPW = B // NW
G = 4
NIT = BPW // G
CH = 40
NCH = L // CH


def _body(x_hbm, tab_hbm, out_hbm, idx_v, rows_v, sem):
    wid = lax.axis_index("s") * NC + lax.axis_index("c")
    for g in range(G):
        for h in range(0, D, 16):
            rows_v[g, L, pl.ds(h, 16)] = jnp.zeros((16,), jnp.float32)

    def it(i, carry):
        b0 = wid * BPW + i * G
        pltpu.sync_copy(x_hbm.at[pl.ds(b0, G)], idx_v)
        copies = [
            pltpu.async_copy(
                tab_hbm.at[idx_v.at[g, h]], rows_v.at[g, pl.ds(h * CH, CH)], sem
            )
            for g in range(G)
            for h in range(NCH)
        ]
        for c in copies:
            c.wait()
        pltpu.sync_copy(rows_v, out_hbm.at[pl.ds(b0, G)])
        return carry

    lax.fori_loop(0, NIT, it, 0)


@functools.partial(jax.jit, static_argnames=())
def _lookup_pad(x, embed_x):
    mesh = plsc.VectorSubcoreMesh(core_axis_name="c", subcore_axis_name="s")
    f = pl.kernel(
        _body,
        out_type=jax.ShapeDtypeStruct((B, LP, D), jnp.float32),
        mesh=mesh,
        scratch_types=[
            pltpu.VMEM((G, NCH, CH), jnp.int32),
            pltpu.VMEM((G, LP, D), jnp.float32),
            pltpu.SemaphoreType.DMA,
        ],
        compiler_params=pltpu.CompilerParams(use_tc_tiling_on_sc=False),
    )
    return f(x, embed_x)


def kernel(x, y, pad_id, embed_x, embed_y):
    xr = x.astype(jnp.int32).reshape(B, NCH, CH)
    return _lookup_pad(xr, embed_x)


# speedup vs baseline: 1.5956x; 1.0304x over previous
import functools

import jax
import jax.numpy as jnp
from jax import lax
from jax.experimental import pallas as pl
from jax.experimental.pallas import tpu as pltpu
from jax.experimental.pallas import tpu_sc as plsc

B = 4096
L = 200
LP = L + 1
D = 32
NC = 2
NS = 16
NW = NC * NS
BPW = B // NW
G = 4
NIT = BPW // G
CH = 40
NCH = L // CH


def _body(x_hbm, tab_hbm, out_hbm, idx_v, rows0, rows1,
          sem_g0, sem_g1, sem_o0, sem_o1):
    wid = lax.axis_index("s") * NC + lax.axis_index("c")
    wb0 = wid * BPW
    rows = (rows0, rows1)
    sem_g = (sem_g0, sem_g1)
    sem_o = (sem_o0, sem_o1)

    pltpu.sync_copy(x_hbm.at[pl.ds(wb0, BPW)], idx_v)

    for s in range(2):
        for g in range(G):
            for h in range(0, D, 16):
                rows[s][g, L, pl.ds(h, 16)] = jnp.zeros((16,), jnp.float32)

    def fire(i, s):
        for g in range(G):
            bb = i * G + g
            for h in range(NCH):
                pltpu.async_copy(
                    tab_hbm.at[idx_v.at[bb, h]],
                    rows[s].at[g, pl.ds(h * CH, CH)],
                    sem_g[s],
                )

    def drain_gathers(s):
        for g in range(G):
            for h in range(NCH):
                pltpu.make_async_copy(
                    tab_hbm.at[idx_v.at[0, h]],
                    rows[s].at[g, pl.ds(h * CH, CH)],
                    sem_g[s],
                ).wait()

    def start_wb(i, s):
        pltpu.async_copy(rows[s], out_hbm.at[pl.ds(wb0 + i * G, G)], sem_o[s])

    def drain_wb(s):
        pltpu.make_async_copy(rows[s], out_hbm.at[pl.ds(wb0, G)], sem_o[s]).wait()

    fire(0, 0)
    drain_gathers(0)
    start_wb(0, 0)
    fire(1, 1)

    def round_body(r, carry):
        i1 = 2 * r + 1
        drain_gathers(1)
        start_wb(i1, 1)
        drain_wb(0)
        fire(i1 + 1, 0)
        i2 = 2 * r + 2
        drain_gathers(0)
        start_wb(i2, 0)
        drain_wb(1)
        fire(i2 + 1, 1)
        return carry

    lax.fori_loop(0, (NIT - 2) // 2, round_body, 0)

    drain_gathers(1)
    start_wb(NIT - 1, 1)
    drain_wb(0)
    drain_wb(1)


@jax.jit
def _lookup_pad(x, embed_x):
    mesh = plsc.VectorSubcoreMesh(core_axis_name="c", subcore_axis_name="s")
    f = pl.kernel(
        _body,
        out_type=jax.ShapeDtypeStruct((B, LP, D), jnp.float32),
        mesh=mesh,
        scratch_types=[
            pltpu.VMEM((BPW, NCH, CH), jnp.int32),
            pltpu.VMEM((G, LP, D), jnp.float32),
            pltpu.VMEM((G, LP, D), jnp.float32),
            pltpu.SemaphoreType.DMA,
            pltpu.SemaphoreType.DMA,
            pltpu.SemaphoreType.DMA,
            pltpu.SemaphoreType.DMA,
        ],
        compiler_params=pltpu.CompilerParams(use_tc_tiling_on_sc=False),
    )
    return f(x, embed_x)


def kernel(x, y, pad_id, embed_x, embed_y):
    xr = x.astype(jnp.int32).reshape(B, NCH, CH)
    return _lookup_pad(xr, embed_x)
